# Initial kernel scaffold; baseline (speedup 1.0000x reference)
import jax
import jax.numpy as jnp
from jax.experimental import pallas as pl


def kernel(x, edge_h, edge_index, n_q_W, n_q_b, n_k_W, n_k_b, n_v_W, n_v_b, e_q_W, e_q_b, e_k_W, e_k_b, e_v_W, e_v_b, tm_n_W, tm_n_b, tm_e_W, tm_e_b, n_lin_W, n_lin_b, Wnd_W, Wnd_b, Wed_W, Wed_b):
    raise NotImplementedError("write your pallas kernel here")


# speedup vs baseline: 36.4927x; 36.4927x over previous
import functools

import jax
import jax.numpy as jnp
from jax import lax
from jax.experimental import pallas as pl
from jax.experimental.pallas import tpu as pltpu
from jax.experimental.pallas import tpu_sc as plsc

N = 10000
E = 320000
D = 128
H = 8
DK = 16
INV_SQRT_DK = 0.25

NC = 2
NS = 16
NW = NC * NS
EPW = E // NW
CH = 80
NCHUNK = EPW // CH

_mesh = plsc.VectorSubcoreMesh(core_axis_name="c", subcore_axis_name="s")


def _node_tables_body(x, wq, bq, wvb, cvb, qn_o, t_o):
    xv = x[...]
    qn_o[...] = jnp.dot(xv, wq[...], preferred_element_type=jnp.float32) + bq[...]
    t_o[:, :D] = xv
    t_o[:, D:] = jnp.dot(xv, wvb[...], preferred_element_type=jnp.float32) + cvb[...]


@functools.partial(
    pl.kernel,
    mesh=_mesh,
    out_type=[
        jax.ShapeDtypeStruct((E, 2 * D), jnp.float32),
        jax.ShapeDtypeStruct((E, D), jnp.float32),
    ],
    scratch_types=[
        pltpu.VMEM((NCHUNK, CH), jnp.int32),
        pltpu.VMEM((NCHUNK, CH), jnp.int32),
        pltpu.VMEM((CH, 2 * D), jnp.float32),
        pltpu.VMEM((CH, D), jnp.float32),
        pltpu.SemaphoreType.DMA,
        pltpu.SemaphoreType.DMA,
    ],
)
def _gather_k(t_hbm, qn_hbm, src_hbm, dst_hbm, gsrc_hbm, gdst_hbm,
              sidx, didx, buf_t, buf_q, sem_t, sem_q):
    c = lax.axis_index("c")
    s = lax.axis_index("s")
    wid = s * NC + c
    base = wid * EPW
    pltpu.sync_copy(src_hbm.at[wid], sidx)
    pltpu.sync_copy(dst_hbm.at[wid], didx)

    def body(j, carry):
        cp_t = pltpu.async_copy(t_hbm.at[sidx.at[j]], buf_t, sem_t)
        cp_q = pltpu.async_copy(qn_hbm.at[didx.at[j]], buf_q, sem_q)
        cp_t.wait()
        cp_q.wait()
        pltpu.sync_copy(buf_t, gsrc_hbm.at[pl.ds(base + j * CH, CH)])
        pltpu.sync_copy(buf_q, gdst_hbm.at[pl.ds(base + j * CH, CH)])
        return carry

    lax.fori_loop(0, NCHUNK, body, 0)


def _edge_body(eh, gs, gd, wk, bk, wv, bv, a2, s16, st16, he_o, pv_o, p_o):
    ehv = eh[...]
    gsv = gs[...]
    ein = ehv + gsv[:, :D]
    ke = jnp.dot(ein, wk[...], preferred_element_type=jnp.float32) + bk[...]
    ve = jnp.dot(ein, wv[...], preferred_element_type=jnp.float32) + bv[...]
    att = jnp.dot(gd[...] * ke, s16[...],
                  preferred_element_type=jnp.float32) * INV_SQRT_DK
    lane = lax.broadcasted_iota(jnp.int32, att.shape, 1)
    p = jnp.where(lane < H, jnp.exp(att), 0.0)
    pb = jnp.dot(p, st16[...], preferred_element_type=jnp.float32)
    p_o[...] = pb
    pv_o[...] = ve * pb
    he = jnp.dot(ehv, a2[...], preferred_element_type=jnp.float32) + gsv[:, D:]
    he_o[...] = jnp.where(he >= 0.0, he, 0.01 * he)


NROWCH = 80
NNCH = N // NROWCH
NNCH_PER_TILE = -(-NNCH // NS)


@functools.partial(
    pl.kernel,
    mesh=_mesh,
    out_type=jax.ShapeDtypeStruct((NC, N, D), jnp.float32),
    scratch_types=[
        pltpu.VMEM((NCHUNK, CH), jnp.int32),
        pltpu.VMEM((NNCH, NROWCH), jnp.int32),
        pltpu.VMEM((CH, D), jnp.float32),
        pltpu.VMEM_SHARED((N, D), jnp.float32),
        pltpu.SemaphoreType.DMA,
    ],
)
def _scatter_k(upd_hbm, dst_hbm, rows_hbm, z128_hbm, acc_hbm,
               didx, ridx, buf_v, acc_sh, sem_v):
    c = lax.axis_index("c")
    s = lax.axis_index("s")
    wid = s * NC + c
    base = wid * EPW

    pltpu.sync_copy(rows_hbm, ridx)
    pltpu.sync_copy(z128_hbm, buf_v)

    def zbody(k, carry):
        j = s + k * NS

        @pl.when(j < NNCH)
        def _():
            pltpu.sync_copy(buf_v, acc_sh.at[ridx.at[j]])

        return carry

    lax.fori_loop(0, NNCH_PER_TILE, zbody, 0)
    plsc.subcore_barrier()
    pltpu.sync_copy(dst_hbm.at[wid], didx)

    def body(j, carry):
        pltpu.sync_copy(upd_hbm.at[pl.ds(base + j * CH, CH)], buf_v)
        pltpu.sync_copy(buf_v, acc_sh.at[didx.at[j]], add=True)
        return carry

    lax.fori_loop(0, NCHUNK, body, 0)
    plsc.subcore_barrier()

    def dbody(k, carry):
        j = s + k * NS

        @pl.when(j < NNCH)
        def _():
            pltpu.async_copy(acc_sh.at[ridx.at[j]], buf_v, sem_v).wait()
            pltpu.sync_copy(buf_v, acc_hbm.at[c, pl.ds(j * NROWCH, NROWCH)])

        return carry

    lax.fori_loop(0, NNCH_PER_TILE, dbody, 0)


def _node_out_body(x, maccp, denp, a1, b1, c1, hn_o):
    macc = maccp[0] + maccp[1]
    den = denp[0] + denp[1]
    m = macc / (den + 1e-9)
    hn = (jnp.dot(x[...], a1[...], preferred_element_type=jnp.float32)
          + jnp.dot(m, b1[...], preferred_element_type=jnp.float32) + c1[...])
    hn_o[...] = jnp.where(hn >= 0.0, hn, 0.01 * hn)


def kernel(x, edge_h, edge_index, n_q_W, n_q_b, n_k_W, n_k_b, n_v_W, n_v_b,
           e_q_W, e_q_b, e_k_W, e_k_b, e_v_W, e_v_b,
           tm_n_W, tm_n_b, tm_e_W, tm_e_b, n_lin_W, n_lin_b,
           Wnd_W, Wnd_b, Wed_W, Wed_b):
    f32 = jnp.float32

    wq_n = n_q_W @ tm_n_W
    bq_n = (n_q_b @ tm_n_W + tm_n_b).reshape(1, D)
    wv_n = n_v_W @ tm_n_W
    bv_n = n_v_b @ tm_n_W + tm_n_b
    b2 = n_lin_W @ Wed_W[D:]
    c2 = n_lin_b @ Wed_W[D:] + Wed_b
    wvb = wv_n @ b2
    cvb = (bv_n @ b2 + c2).reshape(1, D)
    a2 = Wed_W[:D]
    wk_e = e_k_W @ tm_e_W
    bk_e = (e_k_b @ tm_e_W + tm_e_b).reshape(1, D)
    wv_e = e_v_W @ tm_e_W
    bv_e = (e_v_b @ tm_e_W + tm_e_b).reshape(1, D)
    b1 = n_lin_W @ Wnd_W[D:]
    c1 = (n_lin_b @ Wnd_W[D:] + Wnd_b).reshape(1, D)
    a1 = Wnd_W[:D]

    jj = jnp.arange(D)
    s16 = (jj[:, None] // DK == jnp.arange(16)[None, :]).astype(f32)
    st16 = s16.T

    src3 = edge_index[0].reshape(NW, NCHUNK, CH)
    dst3 = edge_index[1].reshape(NW, NCHUNK, CH)

    bn = 1000
    qn, t = pl.pallas_call(
        _node_tables_body,
        grid=(N // bn,),
        in_specs=[
            pl.BlockSpec((bn, D), lambda i: (i, 0)),
            pl.BlockSpec((D, D), lambda i: (0, 0)),
            pl.BlockSpec((1, D), lambda i: (0, 0)),
            pl.BlockSpec((D, D), lambda i: (0, 0)),
            pl.BlockSpec((1, D), lambda i: (0, 0)),
        ],
        out_specs=[
            pl.BlockSpec((bn, D), lambda i: (i, 0)),
            pl.BlockSpec((bn, 2 * D), lambda i: (i, 0)),
        ],
        out_shape=[
            jax.ShapeDtypeStruct((N, D), f32),
            jax.ShapeDtypeStruct((N, 2 * D), f32),
        ],
    )(x, wq_n, bq_n, wvb, cvb)

    gsrc, gdst = _gather_k(t, qn, src3, dst3)

    be = 2000
    h_e, pv, pb = pl.pallas_call(
        _edge_body,
        grid=(E // be,),
        in_specs=[
            pl.BlockSpec((be, D), lambda i: (i, 0)),
            pl.BlockSpec((be, 2 * D), lambda i: (i, 0)),
            pl.BlockSpec((be, D), lambda i: (i, 0)),
            pl.BlockSpec((D, D), lambda i: (0, 0)),
            pl.BlockSpec((1, D), lambda i: (0, 0)),
            pl.BlockSpec((D, D), lambda i: (0, 0)),
            pl.BlockSpec((1, D), lambda i: (0, 0)),
            pl.BlockSpec((D, D), lambda i: (0, 0)),
            pl.BlockSpec((D, 16), lambda i: (0, 0)),
            pl.BlockSpec((16, D), lambda i: (0, 0)),
        ],
        out_specs=[
            pl.BlockSpec((be, D), lambda i: (i, 0)),
            pl.BlockSpec((be, D), lambda i: (i, 0)),
            pl.BlockSpec((be, D), lambda i: (i, 0)),
        ],
        out_shape=[
            jax.ShapeDtypeStruct((E, D), f32),
            jax.ShapeDtypeStruct((E, D), f32),
            jax.ShapeDtypeStruct((E, D), f32),
        ],
    )(edge_h, gsrc, gdst, wk_e, bk_e, wv_e, bv_e, a2, s16, st16)

    z128 = jnp.zeros((NROWCH, D), f32)
    rows = jnp.arange(N, dtype=jnp.int32).reshape(NNCH, NROWCH)
    maccp = _scatter_k(pv, dst3, rows, z128)
    denp = _scatter_k(pb, dst3, rows, z128)

    h_n = pl.pallas_call(
        _node_out_body,
        grid=(N // bn,),
        in_specs=[
            pl.BlockSpec((bn, D), lambda i: (i, 0)),
            pl.BlockSpec((NC, bn, D), lambda i: (0, i, 0)),
            pl.BlockSpec((NC, bn, D), lambda i: (0, i, 0)),
            pl.BlockSpec((D, D), lambda i: (0, 0)),
            pl.BlockSpec((D, D), lambda i: (0, 0)),
            pl.BlockSpec((1, D), lambda i: (0, 0)),
        ],
        out_specs=pl.BlockSpec((bn, D), lambda i: (i, 0)),
        out_shape=jax.ShapeDtypeStruct((N, D), f32),
    )(x, maccp, denp, a1, b1, c1)

    return h_n, h_e


# speedup vs baseline: 38.8659x; 1.0650x over previous
import functools

import jax
import jax.numpy as jnp
from jax import lax
from jax.experimental import pallas as pl
from jax.experimental.pallas import tpu as pltpu
from jax.experimental.pallas import tpu_sc as plsc

N = 10000
E = 320000
D = 128
H = 8
DK = 16
INV_SQRT_DK = 0.25

NC = 2
NS = 16
NW = NC * NS
EPW = E // NW
CH = 80
NCHUNK = EPW // CH

_mesh = plsc.VectorSubcoreMesh(core_axis_name="c", subcore_axis_name="s")


def _node_tables_body(x, wq, bq, wvb, cvb, qn_o, t_o):
    xv = x[...]
    qn_o[...] = jnp.dot(xv, wq[...], preferred_element_type=jnp.float32) + bq[...]
    t_o[:, :D] = xv
    t_o[:, D:] = jnp.dot(xv, wvb[...], preferred_element_type=jnp.float32) + cvb[...]


@functools.partial(
    pl.kernel,
    mesh=_mesh,
    out_type=[
        jax.ShapeDtypeStruct((E, 2 * D), jnp.float32),
        jax.ShapeDtypeStruct((E, D), jnp.float32),
    ],
    scratch_types=[
        pltpu.VMEM((NCHUNK, CH), jnp.int32),
        pltpu.VMEM((NCHUNK, CH), jnp.int32),
        pltpu.VMEM((CH, 2 * D), jnp.float32),
        pltpu.VMEM((CH, D), jnp.float32),
        pltpu.VMEM((CH, 2 * D), jnp.float32),
        pltpu.VMEM((CH, D), jnp.float32),
        pltpu.SemaphoreType.DMA,
        pltpu.SemaphoreType.DMA,
        pltpu.SemaphoreType.DMA,
        pltpu.SemaphoreType.DMA,
    ],
)
def _gather_k(t_hbm, qn_hbm, src_hbm, dst_hbm, gsrc_hbm, gdst_hbm,
              sidx, didx, buf_ta, buf_qa, buf_tb, buf_qb,
              sem_ta, sem_qa, sem_tb, sem_qb):
    c = lax.axis_index("c")
    s = lax.axis_index("s")
    wid = s * NC + c
    base = wid * EPW
    pltpu.sync_copy(src_hbm.at[wid], sidx)
    pltpu.sync_copy(dst_hbm.at[wid], didx)

    def issue(j, bt, bq, st, sq):
        pltpu.async_copy(t_hbm.at[sidx.at[j]], bt, st)
        pltpu.async_copy(qn_hbm.at[didx.at[j]], bq, sq)

    def drain_write(j, bt, bq, st, sq):
        pltpu.make_async_copy(t_hbm.at[sidx.at[j]], bt, st).wait()
        pltpu.make_async_copy(qn_hbm.at[didx.at[j]], bq, sq).wait()
        pltpu.sync_copy(bt, gsrc_hbm.at[pl.ds(base + j * CH, CH)])
        pltpu.sync_copy(bq, gdst_hbm.at[pl.ds(base + j * CH, CH)])

    issue(0, buf_ta, buf_qa, sem_ta, sem_qa)

    def body(i, carry):
        j = 2 * i
        issue(j + 1, buf_tb, buf_qb, sem_tb, sem_qb)
        drain_write(j, buf_ta, buf_qa, sem_ta, sem_qa)
        issue(j + 2, buf_ta, buf_qa, sem_ta, sem_qa)
        drain_write(j + 1, buf_tb, buf_qb, sem_tb, sem_qb)
        return carry

    lax.fori_loop(0, (NCHUNK - 1) // 2, body, 0)
    drain_write(NCHUNK - 1, buf_ta, buf_qa, sem_ta, sem_qa)


def _edge_body(eh, gs, gd, wk, bk, wv, bv, a2, s16, st16, he_o, pv_o, pb_o):
    ehv = eh[...]
    gsv = gs[...]
    ein = ehv + gsv[:, :D]
    ke = jnp.dot(ein, wk[...], preferred_element_type=jnp.float32) + bk[...]
    ve = jnp.dot(ein, wv[...], preferred_element_type=jnp.float32) + bv[...]
    att = jnp.dot(gd[...] * ke, s16[...],
                  preferred_element_type=jnp.float32) * INV_SQRT_DK
    lane = lax.broadcasted_iota(jnp.int32, att.shape, 1)
    p = jnp.where(lane < H, jnp.exp(att), 0.0)
    pb = jnp.dot(p, st16[...], preferred_element_type=jnp.float32)
    pv_o[...] = ve * pb
    pb_o[...] = pb
    he = jnp.dot(ehv, a2[...], preferred_element_type=jnp.float32) + gsv[:, D:]
    he_o[...] = jnp.where(he >= 0.0, he, 0.01 * he)


NROWCH = 80
NNCH = N // NROWCH
NNCH_PER_TILE = -(-NNCH // NS)


@functools.partial(
    pl.kernel,
    mesh=_mesh,
    out_type=jax.ShapeDtypeStruct((NC, N, D), jnp.float32),
    scratch_types=[
        pltpu.VMEM((NCHUNK, CH), jnp.int32),
        pltpu.VMEM((NNCH, NROWCH), jnp.int32),
        pltpu.VMEM((CH, D), jnp.float32),
        pltpu.VMEM_SHARED((N, D), jnp.float32),
        pltpu.SemaphoreType.DMA,
    ],
)
def _scatter_k(upd_hbm, dst_hbm, rows_hbm, z128_hbm, acc_hbm,
               didx, ridx, buf_v, acc_sh, sem_v):
    c = lax.axis_index("c")
    s = lax.axis_index("s")
    wid = s * NC + c
    base = wid * EPW

    pltpu.sync_copy(rows_hbm, ridx)
    pltpu.sync_copy(z128_hbm, buf_v)

    def zbody(k, carry):
        j = s + k * NS

        @pl.when(j < NNCH)
        def _():
            pltpu.sync_copy(buf_v, acc_sh.at[ridx.at[j]])

        return carry

    lax.fori_loop(0, NNCH_PER_TILE, zbody, 0)
    plsc.subcore_barrier()
    pltpu.sync_copy(dst_hbm.at[wid], didx)

    def body(j, carry):
        pltpu.sync_copy(upd_hbm.at[pl.ds(base + j * CH, CH)], buf_v)
        pltpu.sync_copy(buf_v, acc_sh.at[didx.at[j]], add=True)
        return carry

    lax.fori_loop(0, NCHUNK, body, 0)
    plsc.subcore_barrier()

    def dbody(k, carry):
        j = s + k * NS

        @pl.when(j < NNCH)
        def _():
            pltpu.async_copy(acc_sh.at[ridx.at[j]], buf_v, sem_v).wait()
            pltpu.sync_copy(buf_v, acc_hbm.at[c, pl.ds(j * NROWCH, NROWCH)])

        return carry

    lax.fori_loop(0, NNCH_PER_TILE, dbody, 0)


def _node_out_body(x, maccp, denp, a1, b1, c1, hn_o):
    macc = maccp[0] + maccp[1]
    den = denp[0] + denp[1]
    m = macc / (den + 1e-9)
    hn = (jnp.dot(x[...], a1[...], preferred_element_type=jnp.float32)
          + jnp.dot(m, b1[...], preferred_element_type=jnp.float32) + c1[...])
    hn_o[...] = jnp.where(hn >= 0.0, hn, 0.01 * hn)


def kernel(x, edge_h, edge_index, n_q_W, n_q_b, n_k_W, n_k_b, n_v_W, n_v_b,
           e_q_W, e_q_b, e_k_W, e_k_b, e_v_W, e_v_b,
           tm_n_W, tm_n_b, tm_e_W, tm_e_b, n_lin_W, n_lin_b,
           Wnd_W, Wnd_b, Wed_W, Wed_b):
    f32 = jnp.float32

    wq_n = n_q_W @ tm_n_W
    bq_n = (n_q_b @ tm_n_W + tm_n_b).reshape(1, D)
    wv_n = n_v_W @ tm_n_W
    bv_n = n_v_b @ tm_n_W + tm_n_b
    b2 = n_lin_W @ Wed_W[D:]
    c2 = n_lin_b @ Wed_W[D:] + Wed_b
    wvb = wv_n @ b2
    cvb = (bv_n @ b2 + c2).reshape(1, D)
    a2 = Wed_W[:D]
    wk_e = e_k_W @ tm_e_W
    bk_e = (e_k_b @ tm_e_W + tm_e_b).reshape(1, D)
    wv_e = e_v_W @ tm_e_W
    bv_e = (e_v_b @ tm_e_W + tm_e_b).reshape(1, D)
    b1 = n_lin_W @ Wnd_W[D:]
    c1 = (n_lin_b @ Wnd_W[D:] + Wnd_b).reshape(1, D)
    a1 = Wnd_W[:D]

    jj = jnp.arange(D)
    s16 = (jj[:, None] // DK == jnp.arange(16)[None, :]).astype(f32)
    st16 = s16.T

    src3 = edge_index[0].reshape(NW, NCHUNK, CH)
    dst3 = edge_index[1].reshape(NW, NCHUNK, CH)

    bn = 1000
    qn, t = pl.pallas_call(
        _node_tables_body,
        grid=(N // bn,),
        in_specs=[
            pl.BlockSpec((bn, D), lambda i: (i, 0)),
            pl.BlockSpec((D, D), lambda i: (0, 0)),
            pl.BlockSpec((1, D), lambda i: (0, 0)),
            pl.BlockSpec((D, D), lambda i: (0, 0)),
            pl.BlockSpec((1, D), lambda i: (0, 0)),
        ],
        out_specs=[
            pl.BlockSpec((bn, D), lambda i: (i, 0)),
            pl.BlockSpec((bn, 2 * D), lambda i: (i, 0)),
        ],
        out_shape=[
            jax.ShapeDtypeStruct((N, D), f32),
            jax.ShapeDtypeStruct((N, 2 * D), f32),
        ],
    )(x, wq_n, bq_n, wvb, cvb)

    gsrc, gdst = _gather_k(t, qn, src3, dst3)

    be = 2000
    h_e, pv, pb = pl.pallas_call(
        _edge_body,
        grid=(E // be,),
        in_specs=[
            pl.BlockSpec((be, D), lambda i: (i, 0)),
            pl.BlockSpec((be, 2 * D), lambda i: (i, 0)),
            pl.BlockSpec((be, D), lambda i: (i, 0)),
            pl.BlockSpec((D, D), lambda i: (0, 0)),
            pl.BlockSpec((1, D), lambda i: (0, 0)),
            pl.BlockSpec((D, D), lambda i: (0, 0)),
            pl.BlockSpec((1, D), lambda i: (0, 0)),
            pl.BlockSpec((D, D), lambda i: (0, 0)),
            pl.BlockSpec((D, 16), lambda i: (0, 0)),
            pl.BlockSpec((16, D), lambda i: (0, 0)),
        ],
        out_specs=[
            pl.BlockSpec((be, D), lambda i: (i, 0)),
            pl.BlockSpec((be, D), lambda i: (i, 0)),
            pl.BlockSpec((be, D), lambda i: (i, 0)),
        ],
        out_shape=[
            jax.ShapeDtypeStruct((E, D), f32),
            jax.ShapeDtypeStruct((E, D), f32),
            jax.ShapeDtypeStruct((E, D), f32),
        ],
    )(edge_h, gsrc, gdst, wk_e, bk_e, wv_e, bv_e, a2, s16, st16)

    z128 = jnp.zeros((NROWCH, D), f32)
    rows = jnp.arange(N, dtype=jnp.int32).reshape(NNCH, NROWCH)
    maccp = _scatter_k(pv, dst3, rows, z128)
    denp = _scatter_k(pb, dst3, rows, z128)

    h_n = pl.pallas_call(
        _node_out_body,
        grid=(N // bn,),
        in_specs=[
            pl.BlockSpec((bn, D), lambda i: (i, 0)),
            pl.BlockSpec((NC, bn, D), lambda i: (0, i, 0)),
            pl.BlockSpec((NC, bn, D), lambda i: (0, i, 0)),
            pl.BlockSpec((D, D), lambda i: (0, 0)),
            pl.BlockSpec((D, D), lambda i: (0, 0)),
            pl.BlockSpec((1, D), lambda i: (0, 0)),
        ],
        out_specs=pl.BlockSpec((bn, D), lambda i: (i, 0)),
        out_shape=jax.ShapeDtypeStruct((N, D), f32),
    )(x, maccp, denp, a1, b1, c1)

    return h_n, h_e
